# baseline (speedup 1.0000x reference)
import functools

import jax
import jax.numpy as jnp
import numpy as np
from jax import lax
from jax.experimental import pallas as pl
from jax.experimental.pallas import tpu as pltpu
from jax.experimental.pallas import tpu_sc as plsc

_B = 4096
_H = 50
_D = 64
_NE = 999999
_NT = 1000000
_OFF = 499712
_NPAIR = 501760
_NC = 2
_NS = 16
_NW = _NC * _NS
_BPW = _B // _NW
_W = 128
_NWIN = _BPW * _H // _W
_ACC = 2 * _BPW

_SELU_ALPHA = 1.6732632423543772
_SELU_SCALE = 1.0507009873554805


def _tc_build_table(embT):
    cb = 2048
    steps = _NPAIR // cb

    def body(e1_ref, e2_ref, o_ref):
        i = pl.program_id(0)
        left = jnp.transpose(e1_ref[...])
        right = jnp.transpose(e2_ref[...])

        @pl.when(i < steps - 1)
        def _():
            o_ref[...] = jnp.concatenate([left, right], axis=1)

        @pl.when(i == steps - 1)
        def _():
            r = i * cb + lax.broadcasted_iota(jnp.int32, (cb, 1), 0)
            masked = jnp.where(_OFF + r < _NE, right, 0.0)
            o_ref[...] = jnp.concatenate([left, masked], axis=1)

    def snd_map(i):
        return (0, i + _OFF // cb)

    return pl.pallas_call(
        body,
        grid=(steps,),
        in_specs=[
            pl.BlockSpec((_D, cb), lambda i: (0, i)),
            pl.BlockSpec((_D, cb), snd_map),
        ],
        out_specs=pl.BlockSpec((cb, 2 * _D), lambda i: (i, 0)),
        out_shape=jax.ShapeDtypeStruct((_NPAIR, 2 * _D), jnp.float32),
        compiler_params=pltpu.CompilerParams(
            dimension_semantics=("parallel",)),
    )(embT, embT)


def _sc_pool(pairs, idx3d, seg3d):
    mesh = plsc.VectorSubcoreMesh(core_axis_name="c", subcore_axis_name="s",
                                  num_cores=_NC, num_subcores=_NS)

    @functools.partial(
        pl.kernel,
        out_type=jax.ShapeDtypeStruct((_B, _D), jnp.float32),
        mesh=mesh,
        scratch_types=[
            pltpu.VMEM((_NWIN, _W), jnp.int32),
            pltpu.VMEM((_NWIN, _W), jnp.int32),
            pltpu.VMEM((_W, 2 * _D), jnp.float32),
            pltpu.VMEM((_W, 2 * _D), jnp.float32),
            pltpu.VMEM((_ACC, 2 * _D), jnp.float32),
            pltpu.VMEM((_BPW, _D), jnp.float32),
            pltpu.VMEM_SHARED((_NS * _ACC, 2 * _D), jnp.float32),
            pltpu.SemaphoreType.DMA,
            pltpu.SemaphoreType.DMA,
        ],
    )
    def k(pairs_hbm, idx_hbm, seg_hbm, out_hbm,
          idx_v, seg_v, rows_a, rows_b, pair_v, pool_v, acc_sh, sem_a, sem_b):
        cid = lax.axis_index("c")
        sid = lax.axis_index("s")
        wid = sid * _NC + cid
        base = sid * _ACC
        pltpu.sync_copy(idx_hbm.at[wid], idx_v)
        pltpu.sync_copy(seg_hbm.at[wid], seg_v)

        @pl.loop(0, _ACC)
        def _(r):
            for c in range(0, 2 * _D, 16):
                pair_v[r, pl.ds(c, 16)] = jnp.zeros((16,), jnp.float32)
        pltpu.sync_copy(pair_v, acc_sh.at[pl.ds(base, _ACC)])

        def start(w, buf, sem):
            pltpu.async_copy(pairs_hbm.at[idx_v.at[w]], buf, sem)

        def wait(buf, sem):
            pltpu.make_async_copy(pairs_hbm.at[pl.ds(0, _W)], buf, sem).wait()

        def scat(w, buf):
            pltpu.sync_copy(buf, acc_sh.at[seg_v.at[w]], add=True)

        start(0, rows_a, sem_a)

        @pl.loop(0, _NWIN // 2 - 1)
        def _(t):
            w = 2 * t
            start(w + 1, rows_b, sem_b)
            wait(rows_a, sem_a)
            scat(w, rows_a)
            start(w + 2, rows_a, sem_a)
            wait(rows_b, sem_b)
            scat(w + 1, rows_b)

        start(_NWIN - 1, rows_b, sem_b)
        wait(rows_a, sem_a)
        scat(_NWIN - 2, rows_a)
        wait(rows_b, sem_b)
        scat(_NWIN - 1, rows_b)

        pltpu.sync_copy(acc_sh.at[pl.ds(base, _ACC)], pair_v)

        @pl.loop(0, _BPW)
        def _(r):
            for c in range(0, _D, 16):
                pool_v[r, pl.ds(c, 16)] = (
                    pair_v[2 * r, pl.ds(c, 16)]
                    + pair_v[2 * r + 1, pl.ds(_D + c, 16)])

        pltpu.sync_copy(pool_v, out_hbm.at[pl.ds(wid * _BPW, _BPW)])

    return k(pairs, idx3d, seg3d)


def _selu(x):
    return _SELU_SCALE * jnp.where(x > 0, x, _SELU_ALPHA * (jnp.exp(x) - 1.0))


def _tc_mlp(pooled, W1, b1, W2, b2, W3, b3):
    blk = 512
    hi = None

    def body(p_ref, w1_ref, b1_ref, w2_ref, b2_ref, w3_ref, b3_ref, o_ref):
        x = p_ref[...]
        sq = jnp.sum(x * x, axis=-1, keepdims=True)
        x = x * lax.rsqrt(jnp.maximum(sq, 1e-4))
        h = _selu(jnp.dot(x, w1_ref[...], precision=hi) + b1_ref[...])
        h = _selu(jnp.dot(h, w2_ref[...], precision=hi) + b2_ref[...])
        o_ref[...] = jnp.dot(h, w3_ref[...], precision=hi) + b3_ref[...]

    return pl.pallas_call(
        body,
        grid=(_B // blk,),
        in_specs=[
            pl.BlockSpec((blk, _D), lambda i: (i, 0)),
            pl.BlockSpec((_D, 2 * _D), lambda i: (0, 0)),
            pl.BlockSpec((1, 2 * _D), lambda i: (0, 0)),
            pl.BlockSpec((2 * _D, 4 * _D), lambda i: (0, 0)),
            pl.BlockSpec((1, 4 * _D), lambda i: (0, 0)),
            pl.BlockSpec((4 * _D, _D), lambda i: (0, 0)),
            pl.BlockSpec((1, _D), lambda i: (0, 0)),
        ],
        out_specs=pl.BlockSpec((blk, _D), lambda i: (i, 0)),
        out_shape=jax.ShapeDtypeStruct((_B, _D), jnp.float32),
        compiler_params=pltpu.CompilerParams(
            dimension_semantics=("parallel",)),
    )(pooled, W1, b1, W2, b2, W3, b3)


def kernel(inputs, embeddings, W1, b1, W2, b2, W3, b3):
    inputs = inputs.astype(jnp.int32)
    pairs = _tc_build_table(jnp.transpose(embeddings))
    m = jnp.where(inputs == 0, _NT - 1, inputs - 1)
    half = (m >= _OFF).astype(jnp.int32)
    idx3d = (m - _OFF * half).reshape(_NW, _NWIN, _W)
    flat = np.arange(_NW * _NWIN * _W)
    local = (flat % (_BPW * _H)) // _H
    sidv = (flat // (_BPW * _H)) // _NC
    seg_base = jnp.asarray(
        (sidv * _ACC + 2 * local).reshape(_NW, _NWIN, _W).astype(np.int32))
    seg3d = seg_base + half.reshape(_NW, _NWIN, _W)
    pooled = _sc_pool(pairs, idx3d, seg3d)
    return _tc_mlp(pooled, W1, b1.reshape(1, -1), W2, b2.reshape(1, -1),
                   W3, b3.reshape(1, -1))


# speedup vs baseline: 1.3355x; 1.3355x over previous
import functools

import jax
import jax.numpy as jnp
import numpy as np
from jax import lax
from jax.experimental import pallas as pl
from jax.experimental.pallas import tpu as pltpu
from jax.experimental.pallas import tpu_sc as plsc

_B = 4096
_H = 50
_D = 64
_NE = 999999
_NT = 1000000
_OFF = 491520
_NPAIR = 524288
_NC = 2
_NS = 16
_NW = _NC * _NS
_BPW = _B // _NW
_W = 128
_NWIN = _BPW * _H // _W
_ACC = 2 * _BPW

_SELU_ALPHA = 1.6732632423543772
_SELU_SCALE = 1.0507009873554805


def _tc_build_table(embT):
    cb = 16384
    steps = _NPAIR // cb

    def body(e1_ref, e2_ref, o_ref):
        i = pl.program_id(0)
        left = jnp.transpose(e1_ref[...])
        right = jnp.transpose(e2_ref[...])

        @pl.when(i < steps - 1)
        def _():
            o_ref[...] = jnp.concatenate([left, right], axis=1)

        @pl.when(i == steps - 1)
        def _():
            r = i * cb + lax.broadcasted_iota(jnp.int32, (cb, 1), 0)
            masked = jnp.where(_OFF + r < _NE, right, 0.0)
            o_ref[...] = jnp.concatenate([left, masked], axis=1)

    def snd_map(i):
        return (0, i + _OFF // cb)

    return pl.pallas_call(
        body,
        grid=(steps,),
        in_specs=[
            pl.BlockSpec((_D, cb), lambda i: (0, i)),
            pl.BlockSpec((_D, cb), snd_map),
        ],
        out_specs=pl.BlockSpec((cb, 2 * _D), lambda i: (i, 0)),
        out_shape=jax.ShapeDtypeStruct((_NPAIR, 2 * _D), jnp.float32),
        compiler_params=pltpu.CompilerParams(
            dimension_semantics=("parallel",)),
    )(embT, embT)


def _sc_pool(pairs, idx3d, seg3d):
    mesh = plsc.VectorSubcoreMesh(core_axis_name="c", subcore_axis_name="s",
                                  num_cores=_NC, num_subcores=_NS)

    @functools.partial(
        pl.kernel,
        out_type=jax.ShapeDtypeStruct((_B, _D), jnp.float32),
        mesh=mesh,
        scratch_types=[
            pltpu.VMEM((_NWIN, _W), jnp.int32),
            pltpu.VMEM((_NWIN, _W), jnp.int32),
            pltpu.VMEM((_W, 2 * _D), jnp.float32),
            pltpu.VMEM((_W, 2 * _D), jnp.float32),
            pltpu.VMEM((_ACC, 2 * _D), jnp.float32),
            pltpu.VMEM((_BPW, _D), jnp.float32),
            pltpu.VMEM_SHARED((_NS * _ACC, 2 * _D), jnp.float32),
            pltpu.SemaphoreType.DMA,
            pltpu.SemaphoreType.DMA,
        ],
    )
    def k(pairs_hbm, idx_hbm, seg_hbm, out_hbm,
          idx_v, seg_v, rows_a, rows_b, pair_v, pool_v, acc_sh, sem_a, sem_b):
        cid = lax.axis_index("c")
        sid = lax.axis_index("s")
        wid = sid * _NC + cid
        base = sid * _ACC
        pltpu.sync_copy(idx_hbm.at[wid], idx_v)
        pltpu.sync_copy(seg_hbm.at[wid], seg_v)

        @pl.loop(0, _ACC)
        def _(r):
            for c in range(0, 2 * _D, 16):
                pair_v[r, pl.ds(c, 16)] = jnp.zeros((16,), jnp.float32)
        pltpu.sync_copy(pair_v, acc_sh.at[pl.ds(base, _ACC)])

        def start(w, buf, sem):
            pltpu.async_copy(pairs_hbm.at[idx_v.at[w]], buf, sem)

        def wait(buf, sem):
            pltpu.make_async_copy(pairs_hbm.at[pl.ds(0, _W)], buf, sem).wait()

        def scat(w, buf):
            pltpu.sync_copy(buf, acc_sh.at[seg_v.at[w]], add=True)

        start(0, rows_a, sem_a)

        @pl.loop(0, _NWIN // 2 - 1)
        def _(t):
            w = 2 * t
            start(w + 1, rows_b, sem_b)
            wait(rows_a, sem_a)
            scat(w, rows_a)
            start(w + 2, rows_a, sem_a)
            wait(rows_b, sem_b)
            scat(w + 1, rows_b)

        start(_NWIN - 1, rows_b, sem_b)
        wait(rows_a, sem_a)
        scat(_NWIN - 2, rows_a)
        wait(rows_b, sem_b)
        scat(_NWIN - 1, rows_b)

        pltpu.sync_copy(acc_sh.at[pl.ds(base, _ACC)], pair_v)

        @pl.loop(0, _BPW)
        def _(r):
            for c in range(0, _D, 16):
                pool_v[r, pl.ds(c, 16)] = (
                    pair_v[2 * r, pl.ds(c, 16)]
                    + pair_v[2 * r + 1, pl.ds(_D + c, 16)])

        pltpu.sync_copy(pool_v, out_hbm.at[pl.ds(wid * _BPW, _BPW)])

    return k(pairs, idx3d, seg3d)


def _selu(x):
    return _SELU_SCALE * jnp.where(x > 0, x, _SELU_ALPHA * (jnp.exp(x) - 1.0))


def _tc_mlp(pooled, W1, b1, W2, b2, W3, b3):
    blk = 512
    hi = None

    def body(p_ref, w1_ref, b1_ref, w2_ref, b2_ref, w3_ref, b3_ref, o_ref):
        x = p_ref[...]
        sq = jnp.sum(x * x, axis=-1, keepdims=True)
        x = x * lax.rsqrt(jnp.maximum(sq, 1e-4))
        h = _selu(jnp.dot(x, w1_ref[...], precision=hi) + b1_ref[...])
        h = _selu(jnp.dot(h, w2_ref[...], precision=hi) + b2_ref[...])
        o_ref[...] = jnp.dot(h, w3_ref[...], precision=hi) + b3_ref[...]

    return pl.pallas_call(
        body,
        grid=(_B // blk,),
        in_specs=[
            pl.BlockSpec((blk, _D), lambda i: (i, 0)),
            pl.BlockSpec((_D, 2 * _D), lambda i: (0, 0)),
            pl.BlockSpec((1, 2 * _D), lambda i: (0, 0)),
            pl.BlockSpec((2 * _D, 4 * _D), lambda i: (0, 0)),
            pl.BlockSpec((1, 4 * _D), lambda i: (0, 0)),
            pl.BlockSpec((4 * _D, _D), lambda i: (0, 0)),
            pl.BlockSpec((1, _D), lambda i: (0, 0)),
        ],
        out_specs=pl.BlockSpec((blk, _D), lambda i: (i, 0)),
        out_shape=jax.ShapeDtypeStruct((_B, _D), jnp.float32),
        compiler_params=pltpu.CompilerParams(
            dimension_semantics=("parallel",)),
    )(pooled, W1, b1, W2, b2, W3, b3)


def kernel(inputs, embeddings, W1, b1, W2, b2, W3, b3):
    inputs = inputs.astype(jnp.int32)
    pairs = _tc_build_table(jnp.transpose(embeddings))
    m = jnp.where(inputs == 0, _NT - 1, inputs - 1)
    half = (m >= _OFF).astype(jnp.int32)
    idx3d = (m - _OFF * half).reshape(_NW, _NWIN, _W)
    flat = np.arange(_NW * _NWIN * _W)
    local = (flat % (_BPW * _H)) // _H
    sidv = (flat // (_BPW * _H)) // _NC
    seg_base = jnp.asarray(
        (sidv * _ACC + 2 * local).reshape(_NW, _NWIN, _W).astype(np.int32))
    seg3d = seg_base + half.reshape(_NW, _NWIN, _W)
    pooled = _sc_pool(pairs, idx3d, seg3d)
    return _tc_mlp(pooled, W1, b1.reshape(1, -1), W2, b2.reshape(1, -1),
                   W3, b3.reshape(1, -1))


# speedup vs baseline: 1.3426x; 1.0053x over previous
import functools

import jax
import jax.numpy as jnp
import numpy as np
from jax import lax
from jax.experimental import pallas as pl
from jax.experimental.pallas import tpu as pltpu
from jax.experimental.pallas import tpu_sc as plsc

_B = 4096
_H = 50
_D = 64
_NE = 999999
_NT = 1000000
_OFF = 491520
_NPAIR = 524288
_NC = 2
_NS = 16
_NW = _NC * _NS
_BPW = _B // _NW
_W = 128
_NWIN = _BPW * _H // _W
_ACC = 2 * _BPW

_SELU_ALPHA = 1.6732632423543772
_SELU_SCALE = 1.0507009873554805


def _tc_build_table(embT):
    cb = 16384
    steps = _NPAIR // cb

    def body(e1_ref, e2_ref, o_ref):
        i = pl.program_id(0)
        left = jnp.transpose(e1_ref[...])
        right = jnp.transpose(e2_ref[...])

        @pl.when(i < steps - 1)
        def _():
            o_ref[...] = jnp.concatenate([left, right], axis=1)

        @pl.when(i == steps - 1)
        def _():
            r = i * cb + lax.broadcasted_iota(jnp.int32, (cb, 1), 0)
            masked = jnp.where(_OFF + r < _NE, right, 0.0)
            o_ref[...] = jnp.concatenate([left, masked], axis=1)

    def snd_map(i):
        return (0, i + _OFF // cb)

    return pl.pallas_call(
        body,
        grid=(steps,),
        in_specs=[
            pl.BlockSpec((_D, cb), lambda i: (0, i)),
            pl.BlockSpec((_D, cb), snd_map),
        ],
        out_specs=pl.BlockSpec((cb, 2 * _D), lambda i: (i, 0)),
        out_shape=jax.ShapeDtypeStruct((_NPAIR, 2 * _D), jnp.float32),
        compiler_params=pltpu.CompilerParams(
            dimension_semantics=("parallel",)),
    )(embT, embT)


def _sc_pool(pairs, idx3d, seg3d):
    mesh = plsc.VectorSubcoreMesh(core_axis_name="c", subcore_axis_name="s",
                                  num_cores=_NC, num_subcores=_NS)

    @functools.partial(
        pl.kernel,
        out_type=jax.ShapeDtypeStruct((_B, _D), jnp.float32),
        mesh=mesh,
        scratch_types=[
            pltpu.VMEM((_NWIN, _W), jnp.int32),
            pltpu.VMEM((_NWIN, _W), jnp.int32),
            pltpu.VMEM((_W, 2 * _D), jnp.float32),
            pltpu.VMEM((_W, 2 * _D), jnp.float32),
            pltpu.VMEM((_ACC, 2 * _D), jnp.float32),
            pltpu.VMEM((_BPW, _D), jnp.float32),
            pltpu.VMEM_SHARED((_NS * _ACC, 2 * _D), jnp.float32),
            pltpu.SemaphoreType.DMA,
            pltpu.SemaphoreType.DMA,
        ],
    )
    def k(pairs_hbm, idx_hbm, seg_hbm, out_hbm,
          idx_v, seg_v, rows_a, rows_b, pair_v, pool_v, acc_sh, sem_a, sem_b):
        cid = lax.axis_index("c")
        sid = lax.axis_index("s")
        wid = sid * _NC + cid
        base = sid * _ACC
        pltpu.sync_copy(idx_hbm.at[wid], idx_v)
        pltpu.sync_copy(seg_hbm.at[wid], seg_v)

        @pl.loop(0, _ACC)
        def _(r):
            for c in range(0, 2 * _D, 16):
                pair_v[r, pl.ds(c, 16)] = jnp.zeros((16,), jnp.float32)
        pltpu.sync_copy(pair_v, acc_sh.at[pl.ds(base, _ACC)])

        def start(w, buf, sem):
            pltpu.async_copy(pairs_hbm.at[idx_v.at[w]], buf, sem)

        def wait(buf, sem):
            pltpu.make_async_copy(pairs_hbm.at[pl.ds(0, _W)], buf, sem).wait()

        def scat(w, buf):
            pltpu.sync_copy(buf, acc_sh.at[seg_v.at[w]], add=True)

        start(0, rows_a, sem_a)

        @pl.loop(0, _NWIN // 2 - 1)
        def _(t):
            w = 2 * t
            start(w + 1, rows_b, sem_b)
            wait(rows_a, sem_a)
            scat(w, rows_a)
            start(w + 2, rows_a, sem_a)
            wait(rows_b, sem_b)
            scat(w + 1, rows_b)

        start(_NWIN - 1, rows_b, sem_b)
        wait(rows_a, sem_a)
        scat(_NWIN - 2, rows_a)
        wait(rows_b, sem_b)
        scat(_NWIN - 1, rows_b)

        pltpu.sync_copy(acc_sh.at[pl.ds(base, _ACC)], pair_v)

        @pl.loop(0, _BPW)
        def _(r):
            for c in range(0, _D, 16):
                pool_v[r, pl.ds(c, 16)] = (
                    pair_v[2 * r, pl.ds(c, 16)]
                    + pair_v[2 * r + 1, pl.ds(_D + c, 16)])

        pltpu.sync_copy(pool_v, out_hbm.at[pl.ds(wid * _BPW, _BPW)])

    return k(pairs, idx3d, seg3d)


def _selu(x):
    return _SELU_SCALE * jnp.where(x > 0, x, _SELU_ALPHA * (jnp.exp(x) - 1.0))


def _tc_mlp(pooled, W1, b1, W2, b2, W3, b3):
    blk = 1024
    hi = None

    def body(p_ref, w1_ref, b1_ref, w2_ref, b2_ref, w3_ref, b3_ref, o_ref):
        x = p_ref[...]
        sq = jnp.sum(x * x, axis=-1, keepdims=True)
        x = x * lax.rsqrt(jnp.maximum(sq, 1e-4))
        h = _selu(jnp.dot(x, w1_ref[...], precision=hi) + b1_ref[...])
        h = _selu(jnp.dot(h, w2_ref[...], precision=hi) + b2_ref[...])
        o_ref[...] = jnp.dot(h, w3_ref[...], precision=hi) + b3_ref[...]

    return pl.pallas_call(
        body,
        grid=(_B // blk,),
        in_specs=[
            pl.BlockSpec((blk, _D), lambda i: (i, 0)),
            pl.BlockSpec((_D, 2 * _D), lambda i: (0, 0)),
            pl.BlockSpec((1, 2 * _D), lambda i: (0, 0)),
            pl.BlockSpec((2 * _D, 4 * _D), lambda i: (0, 0)),
            pl.BlockSpec((1, 4 * _D), lambda i: (0, 0)),
            pl.BlockSpec((4 * _D, _D), lambda i: (0, 0)),
            pl.BlockSpec((1, _D), lambda i: (0, 0)),
        ],
        out_specs=pl.BlockSpec((blk, _D), lambda i: (i, 0)),
        out_shape=jax.ShapeDtypeStruct((_B, _D), jnp.float32),
        compiler_params=pltpu.CompilerParams(
            dimension_semantics=("parallel",)),
    )(pooled, W1, b1, W2, b2, W3, b3)


def kernel(inputs, embeddings, W1, b1, W2, b2, W3, b3):
    inputs = inputs.astype(jnp.int32)
    pairs = _tc_build_table(jnp.transpose(embeddings))
    m = jnp.where(inputs == 0, _NT - 1, inputs - 1)
    half = (m >= _OFF).astype(jnp.int32)
    idx3d = (m - _OFF * half).reshape(_NW, _NWIN, _W)
    flat = np.arange(_NW * _NWIN * _W)
    local = (flat % (_BPW * _H)) // _H
    sidv = (flat // (_BPW * _H)) // _NC
    seg_base = jnp.asarray(
        (sidv * _ACC + 2 * local).reshape(_NW, _NWIN, _W).astype(np.int32))
    seg3d = seg_base + half.reshape(_NW, _NWIN, _W)
    pooled = _sc_pool(pairs, idx3d, seg3d)
    return _tc_mlp(pooled, W1, b1.reshape(1, -1), W2, b2.reshape(1, -1),
                   W3, b3.reshape(1, -1))


# speedup vs baseline: 1.3685x; 1.0193x over previous
import functools

import jax
import jax.numpy as jnp
import numpy as np
from jax import lax
from jax.experimental import pallas as pl
from jax.experimental.pallas import tpu as pltpu
from jax.experimental.pallas import tpu_sc as plsc

_B = 4096
_H = 50
_D = 64
_NE = 999999
_NT = 1000000
_OFF = 491520
_NPAIR = 524288
_NC = 2
_NS = 16
_NW = _NC * _NS
_BPW = _B // _NW
_W = 128
_NWIN = _BPW * _H // _W
_ACC = 2 * _BPW

_SELU_ALPHA = 1.6732632423543772
_SELU_SCALE = 1.0507009873554805


def _tc_build_table(embT):
    cb = 16384
    steps = _NPAIR // cb

    def body(e1_ref, e2_ref, o_ref):
        i = pl.program_id(0)
        left = jnp.transpose(e1_ref[...])
        right = jnp.transpose(e2_ref[...])

        @pl.when(i < steps - 1)
        def _():
            o_ref[...] = jnp.concatenate([left, right], axis=1)

        @pl.when(i == steps - 1)
        def _():
            r = i * cb + lax.broadcasted_iota(jnp.int32, (cb, 1), 0)
            masked = jnp.where(_OFF + r < _NE, right, 0.0)
            o_ref[...] = jnp.concatenate([left, masked], axis=1)

    def snd_map(i):
        return (0, i + _OFF // cb)

    return pl.pallas_call(
        body,
        grid=(steps,),
        in_specs=[
            pl.BlockSpec((_D, cb), lambda i: (0, i)),
            pl.BlockSpec((_D, cb), snd_map),
        ],
        out_specs=pl.BlockSpec((cb, 2 * _D), lambda i: (i, 0)),
        out_shape=jax.ShapeDtypeStruct((_NPAIR, 2 * _D), jnp.float32),
        compiler_params=pltpu.CompilerParams(
            dimension_semantics=("parallel",)),
    )(embT, embT)


def _sc_pool(pairs, pk3d):
    mesh = plsc.VectorSubcoreMesh(core_axis_name="c", subcore_axis_name="s",
                                  num_cores=_NC, num_subcores=_NS)

    @functools.partial(
        pl.kernel,
        out_type=jax.ShapeDtypeStruct((_B, _D), jnp.float32),
        mesh=mesh,
        scratch_types=[
            pltpu.VMEM((_NWIN, _W), jnp.int32),
            pltpu.VMEM((_NWIN, _W), jnp.int32),
            pltpu.VMEM((_W, 2 * _D), jnp.float32),
            pltpu.VMEM((_W, 2 * _D), jnp.float32),
            pltpu.VMEM((_ACC, 2 * _D), jnp.float32),
            pltpu.VMEM((_BPW, _D), jnp.float32),
            pltpu.VMEM_SHARED((_NS * _ACC, 2 * _D), jnp.float32),
            pltpu.SemaphoreType.DMA,
            pltpu.SemaphoreType.DMA,
        ],
    )
    def k(pairs_hbm, pk_hbm, out_hbm,
          idx_v, seg_v, rows_a, rows_b, pair_v, pool_v, acc_sh, sem_a, sem_b):
        cid = lax.axis_index("c")
        sid = lax.axis_index("s")
        wid = sid * _NC + cid
        base = sid * _ACC
        pltpu.sync_copy(pk_hbm.at[wid], idx_v)

        @pl.loop(0, _NWIN)
        def _(w):
            for c in range(0, _W, 16):
                v = idx_v[w, pl.ds(c, 16)]
                seg_v[w, pl.ds(c, 16)] = lax.shift_right_logical(v, 19)
                idx_v[w, pl.ds(c, 16)] = lax.bitwise_and(
                    v, jnp.full((16,), (1 << 19) - 1, jnp.int32))

        @pl.loop(0, _ACC)
        def _(r):
            for c in range(0, 2 * _D, 16):
                pair_v[r, pl.ds(c, 16)] = jnp.zeros((16,), jnp.float32)
        pltpu.sync_copy(pair_v, acc_sh.at[pl.ds(base, _ACC)])

        def start(w, buf, sem):
            pltpu.async_copy(pairs_hbm.at[idx_v.at[w]], buf, sem)

        def wait(buf, sem):
            pltpu.make_async_copy(pairs_hbm.at[pl.ds(0, _W)], buf, sem).wait()

        def scat(w, buf):
            pltpu.sync_copy(buf, acc_sh.at[seg_v.at[w]], add=True)

        start(0, rows_a, sem_a)

        @pl.loop(0, _NWIN // 2 - 1)
        def _(t):
            w = 2 * t
            start(w + 1, rows_b, sem_b)
            wait(rows_a, sem_a)
            scat(w, rows_a)
            start(w + 2, rows_a, sem_a)
            wait(rows_b, sem_b)
            scat(w + 1, rows_b)

        start(_NWIN - 1, rows_b, sem_b)
        wait(rows_a, sem_a)
        scat(_NWIN - 2, rows_a)
        wait(rows_b, sem_b)
        scat(_NWIN - 1, rows_b)

        pltpu.sync_copy(acc_sh.at[pl.ds(base, _ACC)], pair_v)

        @pl.loop(0, _BPW)
        def _(r):
            for c in range(0, _D, 16):
                pool_v[r, pl.ds(c, 16)] = (
                    pair_v[2 * r, pl.ds(c, 16)]
                    + pair_v[2 * r + 1, pl.ds(_D + c, 16)])

        pltpu.sync_copy(pool_v, out_hbm.at[pl.ds(wid * _BPW, _BPW)])

    return k(pairs, pk3d)


def _selu(x):
    return _SELU_SCALE * jnp.where(x > 0, x, _SELU_ALPHA * (jnp.exp(x) - 1.0))


def _tc_mlp(pooled, W1, b1, W2, b2, W3, b3):
    blk = 1024
    hi = None

    def body(p_ref, w1_ref, b1_ref, w2_ref, b2_ref, w3_ref, b3_ref, o_ref):
        x = p_ref[...]
        sq = jnp.sum(x * x, axis=-1, keepdims=True)
        x = x * lax.rsqrt(jnp.maximum(sq, 1e-4))
        h = _selu(jnp.dot(x, w1_ref[...], precision=hi) + b1_ref[...])
        h = _selu(jnp.dot(h, w2_ref[...], precision=hi) + b2_ref[...])
        o_ref[...] = jnp.dot(h, w3_ref[...], precision=hi) + b3_ref[...]

    return pl.pallas_call(
        body,
        grid=(_B // blk,),
        in_specs=[
            pl.BlockSpec((blk, _D), lambda i: (i, 0)),
            pl.BlockSpec((_D, 2 * _D), lambda i: (0, 0)),
            pl.BlockSpec((1, 2 * _D), lambda i: (0, 0)),
            pl.BlockSpec((2 * _D, 4 * _D), lambda i: (0, 0)),
            pl.BlockSpec((1, 4 * _D), lambda i: (0, 0)),
            pl.BlockSpec((4 * _D, _D), lambda i: (0, 0)),
            pl.BlockSpec((1, _D), lambda i: (0, 0)),
        ],
        out_specs=pl.BlockSpec((blk, _D), lambda i: (i, 0)),
        out_shape=jax.ShapeDtypeStruct((_B, _D), jnp.float32),
        compiler_params=pltpu.CompilerParams(
            dimension_semantics=("parallel",)),
    )(pooled, W1, b1, W2, b2, W3, b3)


def kernel(inputs, embeddings, W1, b1, W2, b2, W3, b3):
    inputs = inputs.astype(jnp.int32)
    pairs = _tc_build_table(jnp.transpose(embeddings))
    m = jnp.where(inputs == 0, _NT - 1, inputs - 1)
    half = (m >= _OFF).astype(jnp.int32)
    q = m - _OFF * half
    b = np.arange(_B)
    seg_base = jnp.asarray(
        (((b // _BPW) // _NC) * _ACC + 2 * (b % _BPW)).astype(np.int32))
    packed = q | ((seg_base[:, None] + half) << 19)
    pooled = _sc_pool(pairs, packed.reshape(_NW, _NWIN, _W))
    return _tc_mlp(pooled, W1, b1.reshape(1, -1), W2, b2.reshape(1, -1),
                   W3, b3.reshape(1, -1))


# speedup vs baseline: 1.3852x; 1.0122x over previous
import functools

import jax
import jax.numpy as jnp
import numpy as np
from jax import lax
from jax.experimental import pallas as pl
from jax.experimental.pallas import tpu as pltpu
from jax.experimental.pallas import tpu_sc as plsc

_B = 4096
_H = 50
_D = 64
_NE = 999999
_NT = 1000000
_OFF = 491520
_NPAIR = 524288
_NC = 2
_NS = 16
_NW = _NC * _NS
_BPW = _B // _NW
_W = 128
_NWIN = _BPW * _H // _W
_ACC = 2 * _BPW

_SELU_ALPHA = 1.6732632423543772
_SELU_SCALE = 1.0507009873554805


def _tc_build_table(embT):
    cb = 16384
    steps = _NPAIR // cb

    def body(e1_ref, e2_ref, o_ref):
        i = pl.program_id(0)
        left = jnp.transpose(e1_ref[...])
        right = jnp.transpose(e2_ref[...])

        @pl.when(i < steps - 1)
        def _():
            o_ref[...] = jnp.concatenate([left, right], axis=1)

        @pl.when(i == steps - 1)
        def _():
            r = i * cb + lax.broadcasted_iota(jnp.int32, (cb, 1), 0)
            masked = jnp.where(_OFF + r < _NE, right, 0.0)
            o_ref[...] = jnp.concatenate([left, masked], axis=1)

    def snd_map(i):
        return (0, i + _OFF // cb)

    return pl.pallas_call(
        body,
        grid=(steps,),
        in_specs=[
            pl.BlockSpec((_D, cb), lambda i: (0, i)),
            pl.BlockSpec((_D, cb), snd_map),
        ],
        out_specs=pl.BlockSpec((cb, 2 * _D), lambda i: (i, 0)),
        out_shape=jax.ShapeDtypeStruct((_NPAIR, 2 * _D), jnp.float32),
        compiler_params=pltpu.CompilerParams(
            dimension_semantics=("parallel",)),
    )(embT, embT)


def _sc_pool(pairs, pk3d):
    mesh = plsc.VectorSubcoreMesh(core_axis_name="c", subcore_axis_name="s",
                                  num_cores=_NC, num_subcores=_NS)

    @functools.partial(
        pl.kernel,
        out_type=jax.ShapeDtypeStruct((_B, _D), jnp.float32),
        mesh=mesh,
        scratch_types=[
            pltpu.VMEM((_NWIN, _W), jnp.int32),
            pltpu.VMEM((_NWIN, _W), jnp.int32),
            pltpu.VMEM((_W, 2 * _D), jnp.float32),
            pltpu.VMEM((_W, 2 * _D), jnp.float32),
            pltpu.VMEM((_ACC, 2 * _D), jnp.float32),
            pltpu.VMEM((_BPW, _D), jnp.float32),
            pltpu.VMEM_SHARED((_NS * _ACC, 2 * _D), jnp.float32),
            pltpu.SemaphoreType.DMA,
            pltpu.SemaphoreType.DMA,
        ],
    )
    def k(pairs_hbm, pk_hbm, out_hbm,
          idx_v, seg_v, rows_a, rows_b, pair_v, pool_v, acc_sh, sem_a, sem_b):
        cid = lax.axis_index("c")
        sid = lax.axis_index("s")
        wid = sid * _NC + cid
        base = sid * _ACC
        pltpu.sync_copy(pk_hbm.at[wid], idx_v)

        @pl.loop(0, _NWIN)
        def _(w):
            for c in range(0, _W, 16):
                v = idx_v[w, pl.ds(c, 16)]
                seg_v[w, pl.ds(c, 16)] = lax.shift_right_logical(v, 19)
                idx_v[w, pl.ds(c, 16)] = lax.bitwise_and(
                    v, jnp.full((16,), (1 << 19) - 1, jnp.int32))

        @pl.loop(0, _ACC)
        def _(r):
            for c in range(0, 2 * _D, 16):
                pair_v[r, pl.ds(c, 16)] = jnp.zeros((16,), jnp.float32)
        pltpu.sync_copy(pair_v, acc_sh.at[pl.ds(base, _ACC)])

        def start(w, buf, sem):
            pltpu.async_copy(pairs_hbm.at[idx_v.at[w]], buf, sem)

        def wait(buf, sem):
            pltpu.make_async_copy(pairs_hbm.at[pl.ds(0, _W)], buf, sem).wait()

        def scat(w, buf):
            pltpu.sync_copy(buf, acc_sh.at[seg_v.at[w]], add=True)

        start(0, rows_a, sem_a)

        @pl.loop(0, _NWIN // 2 - 1)
        def _(t):
            w = 2 * t
            start(w + 1, rows_b, sem_b)
            wait(rows_a, sem_a)
            scat(w, rows_a)
            start(w + 2, rows_a, sem_a)
            wait(rows_b, sem_b)
            scat(w + 1, rows_b)

        start(_NWIN - 1, rows_b, sem_b)
        wait(rows_a, sem_a)
        scat(_NWIN - 2, rows_a)
        wait(rows_b, sem_b)
        scat(_NWIN - 1, rows_b)

        pltpu.sync_copy(acc_sh.at[pl.ds(base, _ACC)], pair_v)

        @pl.loop(0, _BPW)
        def _(r):
            for c in range(0, _D, 16):
                pool_v[r, pl.ds(c, 16)] = (
                    pair_v[2 * r, pl.ds(c, 16)]
                    + pair_v[2 * r + 1, pl.ds(_D + c, 16)])

        pltpu.sync_copy(pool_v, out_hbm.at[pl.ds(wid * _BPW, _BPW)])

    return k(pairs, pk3d)


def _selu(x):
    return _SELU_SCALE * jnp.where(x > 0, x, _SELU_ALPHA * (jnp.exp(x) - 1.0))


def _tc_mlp(pooled, W1, b1, W2, b2, W3, b3):
    blk = 1024
    hi = None

    def body(p_ref, w1_ref, b1_ref, w2_ref, b2_ref, w3_ref, b3_ref, o_ref):
        x = p_ref[...]
        sq = jnp.sum(x * x, axis=-1, keepdims=True)
        x = x * lax.rsqrt(jnp.maximum(sq, 1e-4))
        h = _selu(jnp.dot(x, w1_ref[...], precision=hi) + b1_ref[...])
        h = _selu(jnp.dot(h, w2_ref[...], precision=hi) + b2_ref[...])
        o_ref[...] = jnp.transpose(
            jnp.dot(h, w3_ref[...], precision=hi) + b3_ref[...])

    return pl.pallas_call(
        body,
        grid=(_B // blk,),
        in_specs=[
            pl.BlockSpec((blk, _D), lambda i: (i, 0)),
            pl.BlockSpec((_D, 2 * _D), lambda i: (0, 0)),
            pl.BlockSpec((1, 2 * _D), lambda i: (0, 0)),
            pl.BlockSpec((2 * _D, 4 * _D), lambda i: (0, 0)),
            pl.BlockSpec((1, 4 * _D), lambda i: (0, 0)),
            pl.BlockSpec((4 * _D, _D), lambda i: (0, 0)),
            pl.BlockSpec((1, _D), lambda i: (0, 0)),
        ],
        out_specs=pl.BlockSpec((_D, blk), lambda i: (0, i)),
        out_shape=jax.ShapeDtypeStruct((_D, _B), jnp.float32),
        compiler_params=pltpu.CompilerParams(
            dimension_semantics=("parallel",)),
    )(pooled, W1, b1, W2, b2, W3, b3)


def kernel(inputs, embeddings, W1, b1, W2, b2, W3, b3):
    inputs = inputs.astype(jnp.int32)
    pairs = _tc_build_table(jnp.transpose(embeddings))
    m = jnp.where(inputs == 0, _NT - 1, inputs - 1)
    half = (m >= _OFF).astype(jnp.int32)
    q = m - _OFF * half
    b = np.arange(_B)
    seg_base = jnp.asarray(
        (((b // _BPW) // _NC) * _ACC + 2 * (b % _BPW)).astype(np.int32))
    packed = q | ((seg_base[:, None] + half) << 19)
    pooled = _sc_pool(pairs, packed.reshape(_NW, _NWIN, _W))
    outT = _tc_mlp(pooled, W1, b1.reshape(1, -1), W2, b2.reshape(1, -1),
                   W3, b3.reshape(1, -1))
    return jnp.transpose(outT)
